# baseline (speedup 1.0000x reference)
import functools

import jax
import jax.numpy as jnp
from jax import lax
from jax.experimental import pallas as pl
from jax.experimental.pallas import tpu as pltpu
from jax.experimental.pallas import tpu_sc as plsc

NUM_HEADS = 16
ROWS = 12
L = 16
NC, NS = 2, 16
NW = NC * NS
R = 8
HB = 4
NHB = NUM_HEADS // HB


def _sc_gather_call(N):
    n_slabs = N // R
    per_w = n_slabs // NW
    n_grp = N // L
    tail_col = n_grp * L

    mesh = plsc.VectorSubcoreMesh(
        core_axis_name="c", subcore_axis_name="s",
        num_cores=NC, num_subcores=NS)

    @functools.partial(
        pl.kernel,
        out_type=jax.ShapeDtypeStruct((NUM_HEADS, N, N), jnp.float32),
        mesh=mesh,
        compiler_params=pltpu.CompilerParams(needs_layout_passes=False),
        scratch_types=[
            pltpu.VMEM((2 * 128,), jnp.float32),
            pltpu.VMEM((2, R, N), jnp.int32),
            pltpu.VMEM((2, HB, R, N), jnp.float32),
            pltpu.SemaphoreType.DMA,
            pltpu.SemaphoreType.DMA,
            pltpu.SemaphoreType.DMA,
        ],
    )
    def body(w_hbm, idx_hbm, out_hbm, w_v, idx_v, out_v, sem_idx, s_out0,
             s_out1):
        wid = lax.axis_index("s") * NC + lax.axis_index("c")
        pltpu.sync_copy(w_hbm, w_v)

        lanes = lax.iota(jnp.int32, L)
        rows16 = lanes & (R - 1)
        col_t = jnp.full((L,), tail_col, jnp.int32)
        row_mask = lanes < R
        sems = (s_out0, s_out1)

        pltpu.async_copy(idx_hbm.at[pl.ds(wid * R, R), :], idx_v.at[0],
                         sem_idx)

        def wait_out(b):
            for _ in range(HB):
                pltpu.make_async_copy(out_v.at[b, 0],
                                      out_hbm.at[0, pl.ds(0, R), :],
                                      sems[b]).wait()

        def do_slab(t, carry):
            ib = t & 1
            base = (wid + NW * t) * R
            pltpu.make_async_copy(idx_hbm.at[pl.ds(base, R), :],
                                  idx_v.at[ib], sem_idx).wait()

            @pl.when(t < per_w - 1)
            def _():
                nxt = (wid + NW * (t + 1)) * R
                pltpu.async_copy(idx_hbm.at[pl.ds(nxt, R), :],
                                 idx_v.at[(t + 1) & 1], sem_idx)

            tail_iv = plsc.load_gather(idx_v.at[ib], [rows16, col_t])
            tail_iv = tail_iv * NUM_HEADS

            for hb in range(NHB):
                b = hb & 1
                if hb < 2:
                    @pl.when(t > 0)
                    def _():
                        wait_out(b)
                else:
                    wait_out(b)

                def row_body(r, c2):
                    for g in range(n_grp):
                        iv = idx_v[ib, r, pl.ds(g * L, L)] * NUM_HEADS
                        for j in range(HB):
                            out_v[b, j, r, pl.ds(g * L, L)] = (
                                plsc.load_gather(w_v, [iv + (HB * hb + j)]))
                    return c2
                lax.fori_loop(0, R, row_body, 0)

                for j in range(HB):
                    tv = plsc.load_gather(w_v, [tail_iv + (HB * hb + j)])
                    plsc.store_scatter(out_v.at[b, j], [rows16, col_t], tv,
                                       mask=row_mask)
                for j in range(HB):
                    pltpu.async_copy(out_v.at[b, j],
                                     out_hbm.at[HB * hb + j,
                                                pl.ds(base, R), :],
                                     sems[b])
            return carry

        lax.fori_loop(0, per_w, do_slab, 0)
        wait_out(0)
        wait_out(1)

    return body


def kernel(W, edge_bias):
    N = edge_bias.shape[0]
    call = _sc_gather_call(N)
    w_flat = jnp.pad(W.astype(jnp.float32).reshape(-1),
                     (0, 2 * 128 - ROWS * NUM_HEADS))
    out = call(w_flat, edge_bias.astype(jnp.int32))
    last_row = jnp.take(W.astype(jnp.float32), edge_bias[N - 1], axis=0).T
    out = out.at[:, N - 1, :].set(last_row)
    return out[None]
